# baseline (speedup 1.0000x reference)
import functools

import jax
import jax.numpy as jnp
from jax import lax
from jax.experimental import pallas as pl
from jax.experimental.pallas import tpu as pltpu
from jax.experimental.pallas import tpu_sc as plsc

_N = 10000
_K = 6

_NSC = 2
_NSUB = 16
_NW = _NSC * _NSUB
_EC = 80
_CW = 128
_EPAD = _NW * (_EC + 1) * _CW
_NACC = 10240
_RPT = _NACC // _NSUB
_NHIST = 10240


def _feast_sc_call(zaug, srcg, dstg):
    ch = zaug.shape[1]
    mesh = plsc.VectorSubcoreMesh(core_axis_name="c", subcore_axis_name="s")
    out_type = jax.ShapeDtypeStruct((_NSC, _NACC, ch), jnp.float32)
    scratch = [
        pltpu.VMEM_SHARED((_NACC, ch), jnp.float32),
        pltpu.VMEM((_EC + 1, _CW), jnp.int32),
        pltpu.VMEM((_EC + 1, _CW), jnp.int32),
        pltpu.VMEM((_CW,), jnp.int32),
        pltpu.VMEM((_CW, ch), jnp.float32),
        pltpu.VMEM((_CW, ch), jnp.float32),
        pltpu.VMEM((_RPT, ch), jnp.float32),
        pltpu.SemaphoreType.DMA,
        pltpu.SemaphoreType.DMA,
    ]

    def body(z_h, srcg_h, dstg_h, zw_h, s_out, acc, src_all, dst_all,
             dst2_v, rows0, rows1, zbuf, sem0, sem1):
        c = lax.axis_index("c")
        s = lax.axis_index("s")
        base = s * _RPT
        pltpu.sync_copy(zw_h, zbuf)
        pltpu.sync_copy(zbuf, acc.at[pl.ds(base, _RPT)])
        pltpu.sync_copy(srcg_h.at[c, s], src_all)
        pltpu.sync_copy(dstg_h.at[c, s], dst_all)
        plsc.subcore_barrier()

        def scat(j, rows_v, sem):
            pltpu.make_async_copy(z_h.at[src_all.at[j]], rows_v, sem).wait()
            for l in range(_CW // 16):
                s16 = src_all[j, pl.ds(l * 16, 16)]
                d16 = dst_all[j, pl.ds(l * 16, 16)]
                dst2_v[pl.ds(l * 16, 16)] = jnp.where(
                    s16 == d16, _NACC - 1, d16)
            pltpu.sync_copy(rows_v, acc.at[dst2_v], add=True)

        pltpu.async_copy(z_h.at[src_all.at[0]], rows0, sem0)

        def pair(t, carry):
            j0 = 2 * t
            pltpu.async_copy(z_h.at[src_all.at[j0 + 1]], rows1, sem1)
            scat(j0, rows0, sem0)
            pltpu.async_copy(z_h.at[src_all.at[j0 + 2]], rows0, sem0)
            scat(j0 + 1, rows1, sem1)
            return carry

        lax.fori_loop(0, _EC // 2, pair, 0)
        pltpu.make_async_copy(z_h.at[src_all.at[_EC]], rows0, sem0).wait()
        plsc.subcore_barrier()
        pltpu.sync_copy(acc.at[pl.ds(base, _RPT)],
                        s_out.at[c, pl.ds(base, _RPT)])

    zw = jnp.zeros((_RPT, ch), jnp.float32)
    return pl.kernel(
        body, out_type=out_type, mesh=mesh, scratch_types=scratch,
        compiler_params=pltpu.CompilerParams(use_tc_tiling_on_sc=False),
    )(zaug, srcg, dstg, zw)


def _gather_sc_call(table, idxg, nchunk):
    ch = table.shape[1]
    mesh = plsc.VectorSubcoreMesh(core_axis_name="c", subcore_axis_name="s")
    out_type = jax.ShapeDtypeStruct((_NW, nchunk * _CW, ch), jnp.float32)
    scratch = [
        pltpu.VMEM((nchunk + 1, _CW), jnp.int32),
        pltpu.VMEM((_CW, ch), jnp.float32),
        pltpu.VMEM((_CW, ch), jnp.float32),
        pltpu.SemaphoreType.DMA,
        pltpu.SemaphoreType.DMA,
    ]

    def body(tab_h, idx_h, out_h, idx_all, rows0, rows1, sem0, sem1):
        c = lax.axis_index("c")
        s = lax.axis_index("s")
        wid = c * _NSUB + s
        pltpu.sync_copy(idx_h.at[c, s], idx_all)

        def put(j, rows_v, sem):
            pltpu.make_async_copy(tab_h.at[idx_all.at[j]], rows_v, sem).wait()
            pltpu.sync_copy(rows_v, out_h.at[wid, pl.ds(j * _CW, _CW)])

        pltpu.async_copy(tab_h.at[idx_all.at[0]], rows0, sem0)

        def pair(t, carry):
            j0 = 2 * t
            pltpu.async_copy(tab_h.at[idx_all.at[j0 + 1]], rows1, sem1)
            put(j0, rows0, sem0)
            pltpu.async_copy(tab_h.at[idx_all.at[j0 + 2]], rows0, sem0)
            put(j0 + 1, rows1, sem1)
            return carry

        lax.fori_loop(0, nchunk // 2, pair, 0)
        pltpu.make_async_copy(tab_h.at[idx_all.at[nchunk]], rows0, sem0).wait()

    return pl.kernel(
        body, out_type=out_type, mesh=mesh, scratch_types=scratch,
        compiler_params=pltpu.CompilerParams(use_tc_tiling_on_sc=False),
    )(table, idxg)


def _dense_body(x_ref, w_ref, b_ref, o_ref):
    o_ref[:] = jax.nn.relu(
        jnp.dot(x_ref[:], w_ref[:], preferred_element_type=jnp.float32)
        + b_ref[:])


def _dense_lin_body(x_ref, w_ref, b_ref, o_ref):
    o_ref[:] = (jnp.dot(x_ref[:], w_ref[:], preferred_element_type=jnp.float32)
                + b_ref[:])


def _dense(x, W, b, relu=False, rb=2000):
    n, cin = x.shape
    cout = W.shape[1]
    return pl.pallas_call(
        _dense_body if relu else _dense_lin_body,
        grid=(n // rb,),
        in_specs=[
            pl.BlockSpec((rb, cin), lambda i: (i, 0)),
            pl.BlockSpec((cin, cout), lambda i: (0, 0)),
            pl.BlockSpec((1, cout), lambda i: (0, 0)),
        ],
        out_specs=pl.BlockSpec((rb, cout), lambda i: (i, 0)),
        out_shape=jax.ShapeDtypeStruct((n, cout), jnp.float32),
    )(x, W, b.reshape(1, -1))


def _edge_tail_body(a_ref, bb_ref, g_ref, w2_ref, b2_ref, o_ref):
    a = a_ref[:]
    bb = bb_ref[:]
    w2 = w2_ref[:]
    ch = bb.shape[1]
    acc = None
    for kk in range(_K):
        h = jax.nn.relu(a + g_ref[:, kk * ch:(kk + 1) * ch] - bb)
        hk = jnp.dot(h, w2, preferred_element_type=jnp.float32)
        acc = hk if acc is None else jnp.maximum(acc, hk)
    o_ref[:] = jax.nn.relu(acc + b2_ref[:])


def _edge_tail(a, bb, g, W2, b2):
    ch = bb.shape[1]
    co = W2.shape[1]
    rb = 2000
    return pl.pallas_call(
        _edge_tail_body,
        grid=(_N // rb,),
        in_specs=[
            pl.BlockSpec((rb, ch), lambda i: (i, 0)),
            pl.BlockSpec((rb, ch), lambda i: (i, 0)),
            pl.BlockSpec((rb, _K * ch), lambda i: (i, 0)),
            pl.BlockSpec((ch, co), lambda i: (0, 0)),
            pl.BlockSpec((1, co), lambda i: (0, 0)),
        ],
        out_specs=pl.BlockSpec((rb, co), lambda i: (i, 0)),
        out_shape=jax.ShapeDtypeStruct((_N, co), jnp.float32),
    )(a, bb, g, W2, b2.reshape(1, -1))


def _combine1_body(s0_ref, s1_ref, z_ref, b_ref, o_ref, deg_ref):
    ch = z_ref.shape[1]
    deg = s0_ref[:, ch:ch + 1] + s1_ref[:, ch:ch + 1]
    o_ref[:] = jax.nn.relu(
        (s0_ref[:, :ch] + s1_ref[:, :ch] + z_ref[:]) / (deg + 1.0) + b_ref[:])
    deg_ref[:] = deg


def _feast_combine1(s_parts, z, b):
    ch = z.shape[1]
    cha = s_parts.shape[2]
    rb = 2000
    return pl.pallas_call(
        _combine1_body,
        grid=(_N // rb,),
        in_specs=[
            pl.BlockSpec((rb, cha), lambda i: (i, 0)),
            pl.BlockSpec((rb, cha), lambda i: (i, 0)),
            pl.BlockSpec((rb, ch), lambda i: (i, 0)),
            pl.BlockSpec((1, ch), lambda i: (0, 0)),
        ],
        out_specs=[pl.BlockSpec((rb, ch), lambda i: (i, 0)),
                   pl.BlockSpec((rb, 1), lambda i: (i, 0))],
        out_shape=[jax.ShapeDtypeStruct((_N, ch), jnp.float32),
                   jax.ShapeDtypeStruct((_N, 1), jnp.float32)],
    )(s_parts[0, :_N], s_parts[1, :_N], z, b.reshape(1, -1))


def _combine3_body(s0_ref, s1_ref, z_ref, deg_ref, b_ref, o_ref):
    o_ref[:] = jax.nn.relu(
        (s0_ref[:] + s1_ref[:] + z_ref[:]) / (deg_ref[:] + 1.0) + b_ref[:])


def _feast_combine3(s_parts, z, deg, b):
    ch = z.shape[1]
    rb = 2000
    return pl.pallas_call(
        _combine3_body,
        grid=(_N // rb,),
        in_specs=[
            pl.BlockSpec((rb, ch), lambda i: (i, 0)),
            pl.BlockSpec((rb, ch), lambda i: (i, 0)),
            pl.BlockSpec((rb, ch), lambda i: (i, 0)),
            pl.BlockSpec((rb, 1), lambda i: (i, 0)),
            pl.BlockSpec((1, ch), lambda i: (0, 0)),
        ],
        out_specs=pl.BlockSpec((rb, ch), lambda i: (i, 0)),
        out_shape=jax.ShapeDtypeStruct((_N, ch), jnp.float32),
    )(s_parts[0, :_N], s_parts[1, :_N], z, deg, b.reshape(1, -1))


_NP = 10240
_KNN_RB = 256
_BIG = 3e38


def _knn_body(xr_ref, xsT_ref, out_ref, dscr):
    i = pl.program_id(0)
    rb, np_ = dscr.shape
    xsT = xsT_ref[:]
    sqc = jnp.sum(xsT * xsT, axis=0, keepdims=True)
    d = sqc - 2.0 * jnp.dot(xr_ref[:], xsT, preferred_element_type=jnp.float32)
    rows = jax.lax.broadcasted_iota(jnp.int32, (rb, np_), 0) + i * rb
    cols = jax.lax.broadcasted_iota(jnp.int32, (rb, np_), 1)
    d = jnp.where((rows == cols) | (cols >= _N), _BIG, d)
    dscr[:] = d
    m = jnp.min(d, axis=1, keepdims=True)
    for k in range(_K):
        d = dscr[:]
        am = jnp.min(jnp.where(d <= m, cols, np_), axis=1, keepdims=True)
        out_ref[:, pl.ds(k, 1)] = am
        if k + 1 < _K:
            upd = jnp.where(cols == am, _BIG, d)
            dscr[:] = upd
            m = jnp.min(upd, axis=1, keepdims=True)


def _knn_idx(x, k):
    del k
    c = x.shape[1]
    xp = jnp.pad(x, ((0, _NP - _N), (0, 0)))
    idx = pl.pallas_call(
        _knn_body,
        grid=(_NP // _KNN_RB,),
        in_specs=[
            pl.BlockSpec((_KNN_RB, c), lambda i: (i, 0)),
            pl.BlockSpec((c, _NP), lambda i: (0, 0)),
        ],
        out_specs=pl.BlockSpec((_KNN_RB, 8), lambda i: (i, 0)),
        out_shape=jax.ShapeDtypeStruct((_NP, 8), jnp.int32),
        scratch_shapes=[pltpu.VMEM((_KNN_RB, _NP), jnp.float32)],
    )(xp, xp.T)
    return idx[:_N, :_K]


_GC = 16


def _dyn_edge_conv(x, W1, b1, W2, b2, k):
    del k
    cin = x.shape[1]
    nbr = _knn_idx(x, None)
    co = W1.shape[1]
    wcat = jnp.concatenate([W1[:cin], W1[cin:]], axis=1)
    bcat = jnp.concatenate([b1, jnp.zeros((co,), jnp.float32)])
    abb = _dense(x, wcat, bcat)
    a, bb = abb[:, :co], abb[:, co:]
    idx = nbr.reshape(-1)
    idxg = jnp.pad(idx, (0, _NW * _GC * _CW - idx.shape[0]))
    idxg = jnp.pad(idxg.reshape(_NW, _GC, _CW), ((0, 0), (0, 1), (0, 0)))
    idxg = idxg.reshape(_NSC, _NSUB, _GC + 1, _CW)
    g = _gather_sc_call(bb, idxg, _GC)
    g = g.reshape(_NW * _GC * _CW, -1)[:_N * _K].reshape(_N, _K * co)
    return _edge_tail(a, bb, g, W2, b2)


def _mlp_body(x_ref, w1_ref, b1_ref, w2_ref, b2_ref, w3_ref, b3_ref,
              wo_ref, bo_ref, o_ref):
    h = jax.nn.relu(x_ref[:] @ w1_ref[:] + b1_ref[:])
    h = jax.nn.relu(h @ w2_ref[:] + b2_ref[:])
    h = jax.nn.relu(h @ w3_ref[:] + b3_ref[:])
    o_ref[:] = jax.nn.sigmoid(h @ wo_ref[:] + bo_ref[:])


def _mlp_head(cat1, lin1_W, lin1_b, lin2_W, lin2_b, lin3_W, lin3_b, out_W, out_b):
    rb = 2000
    grid = (_N // rb,)
    full = lambda shape: pl.BlockSpec(shape, lambda i: (0, 0))
    return pl.pallas_call(
        _mlp_body,
        grid=grid,
        in_specs=[
            pl.BlockSpec((rb, 96), lambda i: (i, 0)),
            full((96, 96)), full((1, 96)),
            full((96, 32)), full((1, 32)),
            full((32, 8)), full((1, 8)),
            full((8, 1)), full((1, 1)),
        ],
        out_specs=pl.BlockSpec((rb, 1), lambda i: (i, 0)),
        out_shape=jax.ShapeDtypeStruct((_N, 1), jnp.float32),
    )(cat1, lin1_W, lin1_b.reshape(1, -1), lin2_W, lin2_b.reshape(1, -1),
      lin3_W, lin3_b.reshape(1, -1), out_W, out_b.reshape(1, -1))


def kernel(x, edge_index, conv1_W, conv1_u, conv1_c, conv1_b, conv2_W, conv2_u,
           conv2_c, conv2_b, conv3_W, conv3_u, conv3_c, conv3_b, e1_W1, e1_b1,
           e1_W2, e1_b2, e2_W1, e2_b1, e2_W2, e2_b2, lin1_W, lin1_b, lin2_W,
           lin2_b, lin3_W, lin3_b, out_W, out_b):
    src = edge_index[0]
    dst = edge_index[1]
    pad = _NW * _EC * _CW - src.shape[0]

    def _grp(e):
        e = jnp.pad(e, (0, pad)).reshape(_NW, _EC, _CW)
        e = jnp.pad(e, ((0, 0), (0, 1), (0, 0)))
        return e.reshape(_NSC, _NSUB, _EC + 1, _CW)

    srcg = _grp(src)
    dstg = _grp(dst)

    z1 = _dense(x, conv1_W, conv1_b * 0.0)
    zaug1 = jnp.concatenate(
        [z1, jnp.ones((_N, 1), jnp.float32), jnp.zeros((_N, 15), jnp.float32)],
        axis=1)
    s1p = _feast_sc_call(zaug1, srcg, dstg)
    x1, deg = _feast_combine1(s1p, z1, conv1_b)
    y = _dyn_edge_conv(x1, e1_W1, e1_b1, e1_W2, e1_b2, _K)
    cat0 = jnp.concatenate([x1, y], axis=1)
    y2 = _dyn_edge_conv(y, e2_W1, e2_b1, e2_W2, e2_b2, _K)
    z3 = _dense(cat0, conv3_W, conv3_b * 0.0)
    s3p = _feast_sc_call(z3, srcg, dstg)
    x3 = _feast_combine3(s3p, z3, deg, conv3_b)
    cat1 = jnp.concatenate([x3, y2], axis=1)
    return _mlp_head(cat1, lin1_W, lin1_b, lin2_W, lin2_b, lin3_W, lin3_b,
                     out_W, out_b)


# speedup vs baseline: 1.0197x; 1.0197x over previous
import functools

import jax
import jax.numpy as jnp
from jax import lax
from jax.experimental import pallas as pl
from jax.experimental.pallas import tpu as pltpu
from jax.experimental.pallas import tpu_sc as plsc

_N = 10000
_K = 6

_NSC = 2
_NSUB = 16
_NW = _NSC * _NSUB
_EC = 80
_CW = 128
_EPAD = _NW * (_EC + 1) * _CW
_NACC = 10240
_RPT = _NACC // _NSUB
_NHIST = 10240


def _feast_sc_call(zaug, srcg, dstg):
    ch = zaug.shape[1]
    mesh = plsc.VectorSubcoreMesh(core_axis_name="c", subcore_axis_name="s")
    out_type = jax.ShapeDtypeStruct((_NSC, _NACC, ch), jnp.float32)
    scratch = [
        pltpu.VMEM_SHARED((_NACC, ch), jnp.float32),
        pltpu.VMEM((_EC + 1, _CW), jnp.int32),
        pltpu.VMEM((_EC + 1, _CW), jnp.int32),
        pltpu.VMEM((_CW,), jnp.int32),
        pltpu.VMEM((_CW, ch), jnp.float32),
        pltpu.VMEM((_RPT, ch), jnp.float32),
        pltpu.SemaphoreType.DMA,
    ]

    def body(z_h, srcg_h, dstg_h, zw_h, s_out, acc, src_all, dst_all,
             dst2_v, rows_v, zbuf, sem):
        c = lax.axis_index("c")
        s = lax.axis_index("s")
        base = s * _RPT
        pltpu.sync_copy(zw_h, zbuf)
        pltpu.sync_copy(zbuf, acc.at[pl.ds(base, _RPT)])
        pltpu.sync_copy(srcg_h.at[c, s], src_all)
        pltpu.sync_copy(dstg_h.at[c, s], dst_all)
        plsc.subcore_barrier()

        def chunk(j, carry):
            for l in range(_CW // 16):
                s16 = src_all[j, pl.ds(l * 16, 16)]
                d16 = dst_all[j, pl.ds(l * 16, 16)]
                dst2_v[pl.ds(l * 16, 16)] = jnp.where(
                    s16 == d16, _NACC - 1, d16)
            pltpu.async_copy(z_h.at[src_all.at[j]], rows_v, sem).wait()
            pltpu.sync_copy(rows_v, acc.at[dst2_v], add=True)
            return carry

        lax.fori_loop(0, _EC + 1, chunk, 0)
        plsc.subcore_barrier()
        pltpu.sync_copy(acc.at[pl.ds(base, _RPT)],
                        s_out.at[c, pl.ds(base, _RPT)])

    zw = jnp.zeros((_RPT, ch), jnp.float32)
    return pl.kernel(
        body, out_type=out_type, mesh=mesh, scratch_types=scratch,
        compiler_params=pltpu.CompilerParams(use_tc_tiling_on_sc=False),
    )(zaug, srcg, dstg, zw)


def _gather_sc_call(table, idxg, nchunk):
    ch = table.shape[1]
    mesh = plsc.VectorSubcoreMesh(core_axis_name="c", subcore_axis_name="s")
    out_type = jax.ShapeDtypeStruct((_NW, nchunk * _CW, ch), jnp.float32)
    scratch = [
        pltpu.VMEM((nchunk + 1, _CW), jnp.int32),
        pltpu.VMEM((_CW, ch), jnp.float32),
        pltpu.SemaphoreType.DMA,
    ]

    def body(tab_h, idx_h, out_h, idx_all, rows_v, sem):
        c = lax.axis_index("c")
        s = lax.axis_index("s")
        wid = c * _NSUB + s
        pltpu.sync_copy(idx_h.at[c, s], idx_all)

        def chunk(j, carry):
            pltpu.async_copy(tab_h.at[idx_all.at[j]], rows_v, sem).wait()
            pltpu.sync_copy(rows_v, out_h.at[wid, pl.ds(j * _CW, _CW)])
            return carry

        lax.fori_loop(0, nchunk, chunk, 0)

    return pl.kernel(
        body, out_type=out_type, mesh=mesh, scratch_types=scratch,
        compiler_params=pltpu.CompilerParams(use_tc_tiling_on_sc=False),
    )(table, idxg)


def _dense_body(x_ref, w_ref, b_ref, o_ref):
    o_ref[:] = jax.nn.relu(
        jnp.dot(x_ref[:], w_ref[:], preferred_element_type=jnp.float32)
        + b_ref[:])


def _dense_lin_body(x_ref, w_ref, b_ref, o_ref):
    o_ref[:] = (jnp.dot(x_ref[:], w_ref[:], preferred_element_type=jnp.float32)
                + b_ref[:])


def _dense(x, W, b, relu=False, rb=2000):
    n, cin = x.shape
    cout = W.shape[1]
    return pl.pallas_call(
        _dense_body if relu else _dense_lin_body,
        grid=(n // rb,),
        in_specs=[
            pl.BlockSpec((rb, cin), lambda i: (i, 0)),
            pl.BlockSpec((cin, cout), lambda i: (0, 0)),
            pl.BlockSpec((1, cout), lambda i: (0, 0)),
        ],
        out_specs=pl.BlockSpec((rb, cout), lambda i: (i, 0)),
        out_shape=jax.ShapeDtypeStruct((n, cout), jnp.float32),
    )(x, W, b.reshape(1, -1))


def _edge_tail_body(a_ref, bb_ref, g_ref, w2_ref, b2_ref, o_ref):
    a = a_ref[:]
    bb = bb_ref[:]
    w2 = w2_ref[:]
    ch = bb.shape[1]
    acc = None
    for kk in range(_K):
        h = jax.nn.relu(a + g_ref[:, kk * ch:(kk + 1) * ch] - bb)
        hk = jnp.dot(h, w2, preferred_element_type=jnp.float32)
        acc = hk if acc is None else jnp.maximum(acc, hk)
    o_ref[:] = jax.nn.relu(acc + b2_ref[:])


def _edge_tail(a, bb, g, W2, b2):
    ch = bb.shape[1]
    co = W2.shape[1]
    rb = 2000
    return pl.pallas_call(
        _edge_tail_body,
        grid=(_N // rb,),
        in_specs=[
            pl.BlockSpec((rb, ch), lambda i: (i, 0)),
            pl.BlockSpec((rb, ch), lambda i: (i, 0)),
            pl.BlockSpec((rb, _K * ch), lambda i: (i, 0)),
            pl.BlockSpec((ch, co), lambda i: (0, 0)),
            pl.BlockSpec((1, co), lambda i: (0, 0)),
        ],
        out_specs=pl.BlockSpec((rb, co), lambda i: (i, 0)),
        out_shape=jax.ShapeDtypeStruct((_N, co), jnp.float32),
    )(a, bb, g, W2, b2.reshape(1, -1))


def _combine1_body(s0_ref, s1_ref, z_ref, b_ref, o_ref, deg_ref):
    ch = z_ref.shape[1]
    deg = s0_ref[:, ch:ch + 1] + s1_ref[:, ch:ch + 1]
    o_ref[:] = jax.nn.relu(
        (s0_ref[:, :ch] + s1_ref[:, :ch] + z_ref[:]) / (deg + 1.0) + b_ref[:])
    deg_ref[:] = deg


def _feast_combine1(s_parts, z, b):
    ch = z.shape[1]
    cha = s_parts.shape[2]
    rb = 2000
    return pl.pallas_call(
        _combine1_body,
        grid=(_N // rb,),
        in_specs=[
            pl.BlockSpec((rb, cha), lambda i: (i, 0)),
            pl.BlockSpec((rb, cha), lambda i: (i, 0)),
            pl.BlockSpec((rb, ch), lambda i: (i, 0)),
            pl.BlockSpec((1, ch), lambda i: (0, 0)),
        ],
        out_specs=[pl.BlockSpec((rb, ch), lambda i: (i, 0)),
                   pl.BlockSpec((rb, 1), lambda i: (i, 0))],
        out_shape=[jax.ShapeDtypeStruct((_N, ch), jnp.float32),
                   jax.ShapeDtypeStruct((_N, 1), jnp.float32)],
    )(s_parts[0, :_N], s_parts[1, :_N], z, b.reshape(1, -1))


def _combine3_body(s0_ref, s1_ref, z_ref, deg_ref, b_ref, o_ref):
    o_ref[:] = jax.nn.relu(
        (s0_ref[:] + s1_ref[:] + z_ref[:]) / (deg_ref[:] + 1.0) + b_ref[:])


def _feast_combine3(s_parts, z, deg, b):
    ch = z.shape[1]
    rb = 2000
    return pl.pallas_call(
        _combine3_body,
        grid=(_N // rb,),
        in_specs=[
            pl.BlockSpec((rb, ch), lambda i: (i, 0)),
            pl.BlockSpec((rb, ch), lambda i: (i, 0)),
            pl.BlockSpec((rb, ch), lambda i: (i, 0)),
            pl.BlockSpec((rb, 1), lambda i: (i, 0)),
            pl.BlockSpec((1, ch), lambda i: (0, 0)),
        ],
        out_specs=pl.BlockSpec((rb, ch), lambda i: (i, 0)),
        out_shape=jax.ShapeDtypeStruct((_N, ch), jnp.float32),
    )(s_parts[0, :_N], s_parts[1, :_N], z, deg, b.reshape(1, -1))


_NP = 10240
_KNN_RB = 256
_BIG = 3e38


def _knn_body(xr_ref, xsT_ref, out_ref, dscr):
    i = pl.program_id(0)
    rb, np_ = dscr.shape
    xsT = xsT_ref[:]
    sqc = jnp.sum(xsT * xsT, axis=0, keepdims=True)
    d = sqc - 2.0 * jnp.dot(xr_ref[:], xsT, preferred_element_type=jnp.float32)
    rows = jax.lax.broadcasted_iota(jnp.int32, (rb, np_), 0) + i * rb
    cols = jax.lax.broadcasted_iota(jnp.int32, (rb, np_), 1)
    d = jnp.where((rows == cols) | (cols >= _N), _BIG, d)
    dscr[:] = d
    m = jnp.min(d, axis=1, keepdims=True)
    for k in range(_K):
        d = dscr[:]
        am = jnp.min(jnp.where(d <= m, cols, np_), axis=1, keepdims=True)
        out_ref[:, pl.ds(k, 1)] = am
        if k + 1 < _K:
            upd = jnp.where(cols == am, _BIG, d)
            dscr[:] = upd
            m = jnp.min(upd, axis=1, keepdims=True)


def _knn_idx(x, k):
    del k
    c = x.shape[1]
    xp = jnp.pad(x, ((0, _NP - _N), (0, 0)))
    idx = pl.pallas_call(
        _knn_body,
        grid=(_NP // _KNN_RB,),
        in_specs=[
            pl.BlockSpec((_KNN_RB, c), lambda i: (i, 0)),
            pl.BlockSpec((c, _NP), lambda i: (0, 0)),
        ],
        out_specs=pl.BlockSpec((_KNN_RB, 8), lambda i: (i, 0)),
        out_shape=jax.ShapeDtypeStruct((_NP, 8), jnp.int32),
        scratch_shapes=[pltpu.VMEM((_KNN_RB, _NP), jnp.float32)],
    )(xp, xp.T)
    return idx[:_N, :_K]


_GC = 16


def _dyn_edge_conv(x, W1, b1, W2, b2, k):
    del k
    cin = x.shape[1]
    nbr = _knn_idx(x, None)
    co = W1.shape[1]
    wcat = jnp.concatenate([W1[:cin], W1[cin:]], axis=1)
    bcat = jnp.concatenate([b1, jnp.zeros((co,), jnp.float32)])
    abb = _dense(x, wcat, bcat)
    a, bb = abb[:, :co], abb[:, co:]
    idx = nbr.reshape(-1)
    idxg = jnp.pad(idx, (0, _NW * _GC * _CW - idx.shape[0]))
    idxg = jnp.pad(idxg.reshape(_NW, _GC, _CW), ((0, 0), (0, 1), (0, 0)))
    idxg = idxg.reshape(_NSC, _NSUB, _GC + 1, _CW)
    g = _gather_sc_call(bb, idxg, _GC)
    g = g.reshape(_NW * _GC * _CW, -1)[:_N * _K].reshape(_N, _K * co)
    return _edge_tail(a, bb, g, W2, b2)


def _mlp_body(x_ref, w1_ref, b1_ref, w2_ref, b2_ref, w3_ref, b3_ref,
              wo_ref, bo_ref, o_ref):
    h = jax.nn.relu(x_ref[:] @ w1_ref[:] + b1_ref[:])
    h = jax.nn.relu(h @ w2_ref[:] + b2_ref[:])
    h = jax.nn.relu(h @ w3_ref[:] + b3_ref[:])
    o_ref[:] = jax.nn.sigmoid(h @ wo_ref[:] + bo_ref[:])


def _mlp_head(cat1, lin1_W, lin1_b, lin2_W, lin2_b, lin3_W, lin3_b, out_W, out_b):
    rb = 2000
    grid = (_N // rb,)
    full = lambda shape: pl.BlockSpec(shape, lambda i: (0, 0))
    return pl.pallas_call(
        _mlp_body,
        grid=grid,
        in_specs=[
            pl.BlockSpec((rb, 96), lambda i: (i, 0)),
            full((96, 96)), full((1, 96)),
            full((96, 32)), full((1, 32)),
            full((32, 8)), full((1, 8)),
            full((8, 1)), full((1, 1)),
        ],
        out_specs=pl.BlockSpec((rb, 1), lambda i: (i, 0)),
        out_shape=jax.ShapeDtypeStruct((_N, 1), jnp.float32),
    )(cat1, lin1_W, lin1_b.reshape(1, -1), lin2_W, lin2_b.reshape(1, -1),
      lin3_W, lin3_b.reshape(1, -1), out_W, out_b.reshape(1, -1))


def kernel(x, edge_index, conv1_W, conv1_u, conv1_c, conv1_b, conv2_W, conv2_u,
           conv2_c, conv2_b, conv3_W, conv3_u, conv3_c, conv3_b, e1_W1, e1_b1,
           e1_W2, e1_b2, e2_W1, e2_b1, e2_W2, e2_b2, lin1_W, lin1_b, lin2_W,
           lin2_b, lin3_W, lin3_b, out_W, out_b):
    src = edge_index[0]
    dst = edge_index[1]
    pad = _NW * _EC * _CW - src.shape[0]

    def _grp(e):
        e = jnp.pad(e, (0, pad)).reshape(_NW, _EC, _CW)
        e = jnp.pad(e, ((0, 0), (0, 1), (0, 0)))
        return e.reshape(_NSC, _NSUB, _EC + 1, _CW)

    srcg = _grp(src)
    dstg = _grp(dst)

    z1 = _dense(x, conv1_W, conv1_b * 0.0)
    zaug1 = jnp.concatenate(
        [z1, jnp.ones((_N, 1), jnp.float32), jnp.zeros((_N, 15), jnp.float32)],
        axis=1)
    s1p = _feast_sc_call(zaug1, srcg, dstg)
    x1, deg = _feast_combine1(s1p, z1, conv1_b)
    y = _dyn_edge_conv(x1, e1_W1, e1_b1, e1_W2, e1_b2, _K)
    cat0 = jnp.concatenate([x1, y], axis=1)
    y2 = _dyn_edge_conv(y, e2_W1, e2_b1, e2_W2, e2_b2, _K)
    z3 = _dense(cat0, conv3_W, conv3_b * 0.0)
    s3p = _feast_sc_call(z3, srcg, dstg)
    x3 = _feast_combine3(s3p, z3, deg, conv3_b)
    cat1 = jnp.concatenate([x3, y2], axis=1)
    return _mlp_head(cat1, lin1_W, lin1_b, lin2_W, lin2_b, lin3_W, lin3_b,
                     out_W, out_b)


# speedup vs baseline: 1.1148x; 1.0933x over previous
import functools

import jax
import jax.numpy as jnp
from jax import lax
from jax.experimental import pallas as pl
from jax.experimental.pallas import tpu as pltpu
from jax.experimental.pallas import tpu_sc as plsc

_N = 10000
_K = 6

_NSC = 2
_NSUB = 16
_NW = _NSC * _NSUB
_EC = 79
_CW = 128
_EPAD = _NW * _EC * _CW
_NACC = 10240
_RPT = _NACC // _NSUB
_NHIST = 10240


def _feast_sc_call(zaug, srcg, dstg):
    ch = zaug.shape[1]
    mesh = plsc.VectorSubcoreMesh(core_axis_name="c", subcore_axis_name="s")
    out_type = jax.ShapeDtypeStruct((_NSC, _NACC, ch), jnp.float32)
    scratch = [
        pltpu.VMEM_SHARED((_NACC, ch), jnp.float32),
        pltpu.VMEM((_EC, _CW), jnp.int32),
        pltpu.VMEM((_EC, _CW), jnp.int32),
        pltpu.VMEM((_CW,), jnp.int32),
        pltpu.VMEM((_CW, ch), jnp.float32),
        pltpu.VMEM((_RPT, ch), jnp.float32),
        pltpu.SemaphoreType.DMA,
    ]

    def body(z_h, srcg_h, dstg_h, zw_h, s_out, acc, src_all, dst_all,
             dst2_v, rows_v, zbuf, sem):
        c = lax.axis_index("c")
        s = lax.axis_index("s")
        base = s * _RPT
        pltpu.sync_copy(zw_h, zbuf)
        pltpu.sync_copy(zbuf, acc.at[pl.ds(base, _RPT)])
        pltpu.sync_copy(srcg_h.at[c, s], src_all)
        pltpu.sync_copy(dstg_h.at[c, s], dst_all)
        plsc.subcore_barrier()

        def chunk(j, carry):
            for l in range(_CW // 16):
                s16 = src_all[j, pl.ds(l * 16, 16)]
                d16 = dst_all[j, pl.ds(l * 16, 16)]
                dump = _N + l * 16 + lax.iota(jnp.int32, 16)
                dst2_v[pl.ds(l * 16, 16)] = jnp.where(s16 == d16, dump, d16)
            pltpu.async_copy(z_h.at[src_all.at[j]], rows_v, sem).wait()
            pltpu.sync_copy(rows_v, acc.at[dst2_v], add=True)
            return carry

        lax.fori_loop(0, _EC, chunk, 0)
        plsc.subcore_barrier()
        pltpu.sync_copy(acc.at[pl.ds(base, _RPT)],
                        s_out.at[c, pl.ds(base, _RPT)])

    zw = jnp.zeros((_RPT, ch), jnp.float32)
    return pl.kernel(
        body, out_type=out_type, mesh=mesh, scratch_types=scratch,
        compiler_params=pltpu.CompilerParams(use_tc_tiling_on_sc=False),
    )(zaug, srcg, dstg, zw)


def _gather_sc_call(table, idxg, nchunk):
    ch = table.shape[1]
    mesh = plsc.VectorSubcoreMesh(core_axis_name="c", subcore_axis_name="s")
    out_type = jax.ShapeDtypeStruct((_NW, nchunk * _CW, ch), jnp.float32)
    scratch = [
        pltpu.VMEM((nchunk, _CW), jnp.int32),
        pltpu.VMEM((_CW, ch), jnp.float32),
        pltpu.SemaphoreType.DMA,
    ]

    def body(tab_h, idx_h, out_h, idx_all, rows_v, sem):
        c = lax.axis_index("c")
        s = lax.axis_index("s")
        wid = c * _NSUB + s
        pltpu.sync_copy(idx_h.at[c, s], idx_all)

        def chunk(j, carry):
            pltpu.async_copy(tab_h.at[idx_all.at[j]], rows_v, sem).wait()
            pltpu.sync_copy(rows_v, out_h.at[wid, pl.ds(j * _CW, _CW)])
            return carry

        lax.fori_loop(0, nchunk, chunk, 0)

    return pl.kernel(
        body, out_type=out_type, mesh=mesh, scratch_types=scratch,
        compiler_params=pltpu.CompilerParams(use_tc_tiling_on_sc=False),
    )(table, idxg)


def _dense_body(x_ref, w_ref, b_ref, o_ref):
    o_ref[:] = jax.nn.relu(
        jnp.dot(x_ref[:], w_ref[:], preferred_element_type=jnp.float32)
        + b_ref[:])


def _dense_lin_body(x_ref, w_ref, b_ref, o_ref):
    o_ref[:] = (jnp.dot(x_ref[:], w_ref[:], preferred_element_type=jnp.float32)
                + b_ref[:])


def _dense(x, W, b, relu=False, rb=2000):
    n, cin = x.shape
    cout = W.shape[1]
    return pl.pallas_call(
        _dense_body if relu else _dense_lin_body,
        grid=(n // rb,),
        in_specs=[
            pl.BlockSpec((rb, cin), lambda i: (i, 0)),
            pl.BlockSpec((cin, cout), lambda i: (0, 0)),
            pl.BlockSpec((1, cout), lambda i: (0, 0)),
        ],
        out_specs=pl.BlockSpec((rb, cout), lambda i: (i, 0)),
        out_shape=jax.ShapeDtypeStruct((n, cout), jnp.float32),
    )(x, W, b.reshape(1, -1))


def _edge_tail_body(a_ref, bb_ref, g_ref, w2_ref, b2_ref, o_ref):
    a = a_ref[:]
    bb = bb_ref[:]
    w2 = w2_ref[:]
    ch = bb.shape[1]
    acc = None
    for kk in range(_K):
        h = jax.nn.relu(a + g_ref[:, kk * ch:(kk + 1) * ch] - bb)
        hk = jnp.dot(h, w2, preferred_element_type=jnp.float32)
        acc = hk if acc is None else jnp.maximum(acc, hk)
    o_ref[:] = jax.nn.relu(acc + b2_ref[:])


def _edge_tail(a, bb, g, W2, b2):
    ch = bb.shape[1]
    co = W2.shape[1]
    rb = 2000
    return pl.pallas_call(
        _edge_tail_body,
        grid=(_N // rb,),
        in_specs=[
            pl.BlockSpec((rb, ch), lambda i: (i, 0)),
            pl.BlockSpec((rb, ch), lambda i: (i, 0)),
            pl.BlockSpec((rb, _K * ch), lambda i: (i, 0)),
            pl.BlockSpec((ch, co), lambda i: (0, 0)),
            pl.BlockSpec((1, co), lambda i: (0, 0)),
        ],
        out_specs=pl.BlockSpec((rb, co), lambda i: (i, 0)),
        out_shape=jax.ShapeDtypeStruct((_N, co), jnp.float32),
    )(a, bb, g, W2, b2.reshape(1, -1))


def _combine1_body(s0_ref, s1_ref, z_ref, b_ref, o_ref, deg_ref):
    ch = z_ref.shape[1]
    deg = s0_ref[:, ch:ch + 1] + s1_ref[:, ch:ch + 1]
    o_ref[:] = jax.nn.relu(
        (s0_ref[:, :ch] + s1_ref[:, :ch] + z_ref[:]) / (deg + 1.0) + b_ref[:])
    deg_ref[:] = deg


def _feast_combine1(s_parts, z, b):
    ch = z.shape[1]
    cha = s_parts.shape[2]
    rb = 2000
    return pl.pallas_call(
        _combine1_body,
        grid=(_N // rb,),
        in_specs=[
            pl.BlockSpec((rb, cha), lambda i: (i, 0)),
            pl.BlockSpec((rb, cha), lambda i: (i, 0)),
            pl.BlockSpec((rb, ch), lambda i: (i, 0)),
            pl.BlockSpec((1, ch), lambda i: (0, 0)),
        ],
        out_specs=[pl.BlockSpec((rb, ch), lambda i: (i, 0)),
                   pl.BlockSpec((rb, 1), lambda i: (i, 0))],
        out_shape=[jax.ShapeDtypeStruct((_N, ch), jnp.float32),
                   jax.ShapeDtypeStruct((_N, 1), jnp.float32)],
    )(s_parts[0, :_N], s_parts[1, :_N], z, b.reshape(1, -1))


def _combine3_body(s0_ref, s1_ref, z_ref, deg_ref, b_ref, o_ref):
    o_ref[:] = jax.nn.relu(
        (s0_ref[:] + s1_ref[:] + z_ref[:]) / (deg_ref[:] + 1.0) + b_ref[:])


def _feast_combine3(s_parts, z, deg, b):
    ch = z.shape[1]
    rb = 2000
    return pl.pallas_call(
        _combine3_body,
        grid=(_N // rb,),
        in_specs=[
            pl.BlockSpec((rb, ch), lambda i: (i, 0)),
            pl.BlockSpec((rb, ch), lambda i: (i, 0)),
            pl.BlockSpec((rb, ch), lambda i: (i, 0)),
            pl.BlockSpec((rb, 1), lambda i: (i, 0)),
            pl.BlockSpec((1, ch), lambda i: (0, 0)),
        ],
        out_specs=pl.BlockSpec((rb, ch), lambda i: (i, 0)),
        out_shape=jax.ShapeDtypeStruct((_N, ch), jnp.float32),
    )(s_parts[0, :_N], s_parts[1, :_N], z, deg, b.reshape(1, -1))


_NP = 10240
_KNN_RB = 256
_BIG = 3e38


def _knn_body(xr_ref, xsT_ref, out_ref, dscr):
    i = pl.program_id(0)
    rb, np_ = dscr.shape
    xsT = xsT_ref[:]
    sqc = jnp.sum(xsT * xsT, axis=0, keepdims=True)
    d = sqc - 2.0 * jnp.dot(xr_ref[:], xsT, preferred_element_type=jnp.float32)
    rows = jax.lax.broadcasted_iota(jnp.int32, (rb, np_), 0) + i * rb
    cols = jax.lax.broadcasted_iota(jnp.int32, (rb, np_), 1)
    d = jnp.where((rows == cols) | (cols >= _N), _BIG, d)
    dscr[:] = d
    m = jnp.min(d, axis=1, keepdims=True)
    for k in range(_K):
        d = dscr[:]
        am = jnp.min(jnp.where(d <= m, cols, np_), axis=1, keepdims=True)
        out_ref[:, pl.ds(k, 1)] = am
        if k + 1 < _K:
            upd = jnp.where(cols == am, _BIG, d)
            dscr[:] = upd
            m = jnp.min(upd, axis=1, keepdims=True)


def _knn_idx(x, k):
    del k
    c = x.shape[1]
    xp = jnp.pad(x, ((0, _NP - _N), (0, 0)))
    idx = pl.pallas_call(
        _knn_body,
        grid=(_NP // _KNN_RB,),
        in_specs=[
            pl.BlockSpec((_KNN_RB, c), lambda i: (i, 0)),
            pl.BlockSpec((c, _NP), lambda i: (0, 0)),
        ],
        out_specs=pl.BlockSpec((_KNN_RB, 8), lambda i: (i, 0)),
        out_shape=jax.ShapeDtypeStruct((_NP, 8), jnp.int32),
        scratch_shapes=[pltpu.VMEM((_KNN_RB, _NP), jnp.float32)],
    )(xp, xp.T)
    return idx[:_N, :_K]


_GC = 15


def _dyn_edge_conv(x, W1, b1, W2, b2, k):
    del k
    cin = x.shape[1]
    nbr = _knn_idx(x, None)
    co = W1.shape[1]
    wcat = jnp.concatenate([W1[:cin], W1[cin:]], axis=1)
    bcat = jnp.concatenate([b1, jnp.zeros((co,), jnp.float32)])
    abb = _dense(x, wcat, bcat)
    a, bb = abb[:, :co], abb[:, co:]
    idx = nbr.reshape(-1)
    idxg = jnp.pad(idx, (0, _NW * _GC * _CW - idx.shape[0]))
    idxg = idxg.reshape(_NSC, _NSUB, _GC, _CW)
    g = _gather_sc_call(bb, idxg, _GC)
    g = g.reshape(_NW * _GC * _CW, -1)[:_N * _K].reshape(_N, _K * co)
    return _edge_tail(a, bb, g, W2, b2)


def _mlp_body(x_ref, w1_ref, b1_ref, w2_ref, b2_ref, w3_ref, b3_ref,
              wo_ref, bo_ref, o_ref):
    h = jax.nn.relu(x_ref[:] @ w1_ref[:] + b1_ref[:])
    h = jax.nn.relu(h @ w2_ref[:] + b2_ref[:])
    h = jax.nn.relu(h @ w3_ref[:] + b3_ref[:])
    o_ref[:] = jax.nn.sigmoid(h @ wo_ref[:] + bo_ref[:])


def _mlp_head(cat1, lin1_W, lin1_b, lin2_W, lin2_b, lin3_W, lin3_b, out_W, out_b):
    rb = 2000
    grid = (_N // rb,)
    full = lambda shape: pl.BlockSpec(shape, lambda i: (0, 0))
    return pl.pallas_call(
        _mlp_body,
        grid=grid,
        in_specs=[
            pl.BlockSpec((rb, 96), lambda i: (i, 0)),
            full((96, 96)), full((1, 96)),
            full((96, 32)), full((1, 32)),
            full((32, 8)), full((1, 8)),
            full((8, 1)), full((1, 1)),
        ],
        out_specs=pl.BlockSpec((rb, 1), lambda i: (i, 0)),
        out_shape=jax.ShapeDtypeStruct((_N, 1), jnp.float32),
    )(cat1, lin1_W, lin1_b.reshape(1, -1), lin2_W, lin2_b.reshape(1, -1),
      lin3_W, lin3_b.reshape(1, -1), out_W, out_b.reshape(1, -1))


def kernel(x, edge_index, conv1_W, conv1_u, conv1_c, conv1_b, conv2_W, conv2_u,
           conv2_c, conv2_b, conv3_W, conv3_u, conv3_c, conv3_b, e1_W1, e1_b1,
           e1_W2, e1_b2, e2_W1, e2_b1, e2_W2, e2_b2, lin1_W, lin1_b, lin2_W,
           lin2_b, lin3_W, lin3_b, out_W, out_b):
    src = edge_index[0]
    dst = edge_index[1]
    pad = _EPAD - src.shape[0]
    srcg = jnp.pad(src, (0, pad)).reshape(_NSC, _NSUB, _EC, _CW)
    dstg = jnp.pad(dst, (0, pad)).reshape(_NSC, _NSUB, _EC, _CW)

    z1 = _dense(x, conv1_W, conv1_b * 0.0)
    zaug1 = jnp.concatenate(
        [z1, jnp.ones((_N, 1), jnp.float32), jnp.zeros((_N, 15), jnp.float32)],
        axis=1)
    s1p = _feast_sc_call(zaug1, srcg, dstg)
    x1, deg = _feast_combine1(s1p, z1, conv1_b)
    y = _dyn_edge_conv(x1, e1_W1, e1_b1, e1_W2, e1_b2, _K)
    cat0 = jnp.concatenate([x1, y], axis=1)
    y2 = _dyn_edge_conv(y, e2_W1, e2_b1, e2_W2, e2_b2, _K)
    z3 = _dense(cat0, conv3_W, conv3_b * 0.0)
    s3p = _feast_sc_call(z3, srcg, dstg)
    x3 = _feast_combine3(s3p, z3, deg, conv3_b)
    cat1 = jnp.concatenate([x3, y2], axis=1)
    return _mlp_head(cat1, lin1_W, lin1_b, lin2_W, lin2_b, lin3_W, lin3_b,
                     out_W, out_b)
